# baseline (speedup 1.0000x reference)
import functools

import jax
import jax.numpy as jnp
from jax.experimental import pallas as pl
from jax.experimental.pallas import tpu as pltpu


def _roi_body(fm_hbm, cd_ref, sc_ref, out_ref, buf, sems, *, H, W, n_chunk):
    N = sc_ref.shape[2]
    c_chunk = buf.shape[1]
    n_slots = buf.shape[0]
    b = pl.program_id(0)

    cd = cd_ref[0]
    x0 = cd[0:1, :]
    x1 = cd[1:2, :]
    y0 = cd[2:3, :]
    y1 = cd[3:4, :]

    xi = jax.lax.broadcasted_iota(jnp.int32, (W, N), 0)
    colt = jnp.where((xi >= x0) & (xi < x1), 1.0, 0.0).astype(jnp.float32)

    def start(i):
        pltpu.make_async_copy(
            fm_hbm.at[b, pl.ds(i * c_chunk, c_chunk)],
            buf.at[i % n_slots],
            sems.at[i % n_slots],
        ).start()

    for i in range(min(n_slots, n_chunk)):
        start(i)

    for i in range(n_chunk):
        pltpu.make_async_copy(
            fm_hbm.at[b, pl.ds(i * c_chunk, c_chunk)],
            buf.at[i % n_slots],
            sems.at[i % n_slots],
        ).wait()
        fm = buf[i % n_slots]
        acc = jnp.zeros((c_chunk, N), jnp.float32)
        for yc in range(0, H, 8):
            rows = min(8, H - yc)
            xc = fm[:, yc:yc + rows, :].reshape(c_chunk * rows, W)
            uc = jnp.dot(xc, colt, preferred_element_type=jnp.float32)
            uc = uc.reshape(c_chunk, rows, N)
            yi = jax.lax.broadcasted_iota(jnp.int32, (rows, N), 0) + yc
            rc = jnp.where((yi >= y0) & (yi < y1), 1.0, 0.0).astype(jnp.float32)
            acc = acc + jnp.sum(uc * rc[None, :, :], axis=1)
        out_ref[0, i * c_chunk:(i + 1) * c_chunk, :] = acc * sc_ref[0]
        if i + n_slots < n_chunk:
            start(i + n_slots)


def kernel(feature_map, keypoints, mask, original_H, original_W):
    B, C, H, W = feature_map.shape
    N = keypoints.shape[1]
    sx = W / original_W
    sy = H / original_H
    x, y, w, h = (keypoints[..., 0], keypoints[..., 1],
                  keypoints[..., 2], keypoints[..., 3])
    xr = jnp.clip((x * sx).astype(jnp.int32), 0, W - 1)
    yr = jnp.clip((y * sy).astype(jnp.int32), 0, H - 1)
    wr = jnp.minimum(jnp.maximum((w * sx).astype(jnp.int32), 1), W - xr)
    hr = jnp.minimum(jnp.maximum((h * sy).astype(jnp.int32), 1), H - yr)
    coords = jnp.stack([xr, xr + wr, yr, yr + hr], axis=1)
    area = (hr * wr).astype(jnp.float32)
    scale = jnp.where(mask > 0, 1.0 / area, 0.0).reshape(B, 1, N)

    c_chunk = 32
    n_chunk = C // c_chunk
    n_slots = 4
    out = pl.pallas_call(
        functools.partial(_roi_body, H=H, W=W, n_chunk=n_chunk),
        grid=(B,),
        in_specs=[
            pl.BlockSpec(memory_space=pl.ANY),
            pl.BlockSpec((1, 4, N), lambda b: (b, 0, 0)),
            pl.BlockSpec((1, 1, N), lambda b: (b, 0, 0)),
        ],
        out_specs=pl.BlockSpec((1, C, N), lambda b: (b, 0, 0)),
        out_shape=jax.ShapeDtypeStruct((B, C, N), jnp.float32),
        scratch_shapes=[
            pltpu.VMEM((n_slots, c_chunk, H, W), jnp.float32),
            pltpu.SemaphoreType.DMA((n_slots,)),
        ],
        compiler_params=pltpu.CompilerParams(
            dimension_semantics=("parallel",),
            vmem_limit_bytes=50 * 1024 * 1024,
        ),
    )(feature_map, coords, scale)
    return jnp.transpose(out, (0, 2, 1))


# speedup vs baseline: 1.8429x; 1.8429x over previous
import functools

import jax
import jax.numpy as jnp
from jax.experimental import pallas as pl
from jax.experimental.pallas import tpu as pltpu


def _roi_body(fm_ref, cd_ref, sc_ref, out_ref, *, H, W):
    N = sc_ref.shape[2]
    cd = cd_ref[0]
    x0 = cd[0:1, :]
    x1 = cd[1:2, :]
    y0 = cd[2:3, :]
    y1 = cd[3:4, :]

    xi = jax.lax.broadcasted_iota(jnp.int32, (W, N), 0)
    colf = jnp.where((xi >= x0) & (xi < x1), 1.0, 0.0).astype(jnp.float32)
    yi = jax.lax.broadcasted_iota(jnp.int32, (H, N), 0)
    rowf = jnp.where((yi >= y0) & (yi < y1), 1.0, 0.0).astype(jnp.float32)

    m3 = rowf[:, None, :] * colf[None, :, :]
    ind = m3.reshape(H * W, N).astype(jnp.bfloat16)

    acc = jnp.dot(fm_ref[0], ind, preferred_element_type=jnp.float32)
    out_ref[0] = acc * sc_ref[0]


def kernel(feature_map, keypoints, mask, original_H, original_W):
    B, C, H, W = feature_map.shape
    N = keypoints.shape[1]
    sx = W / original_W
    sy = H / original_H
    x, y, w, h = (keypoints[..., 0], keypoints[..., 1],
                  keypoints[..., 2], keypoints[..., 3])
    xr = jnp.clip((x * sx).astype(jnp.int32), 0, W - 1)
    yr = jnp.clip((y * sy).astype(jnp.int32), 0, H - 1)
    wr = jnp.minimum(jnp.maximum((w * sx).astype(jnp.int32), 1), W - xr)
    hr = jnp.minimum(jnp.maximum((h * sy).astype(jnp.int32), 1), H - yr)
    coords = jnp.stack([xr, xr + wr, yr, yr + hr], axis=1)
    area = (hr * wr).astype(jnp.float32)
    scale = jnp.where(mask > 0, 1.0 / area, 0.0).reshape(B, 1, N)

    fm = feature_map.reshape(B, C, H * W).astype(jnp.bfloat16)
    out = pl.pallas_call(
        functools.partial(_roi_body, H=H, W=W),
        grid=(B,),
        in_specs=[
            pl.BlockSpec((1, C, H * W), lambda b: (b, 0, 0)),
            pl.BlockSpec((1, 4, N), lambda b: (b, 0, 0)),
            pl.BlockSpec((1, 1, N), lambda b: (b, 0, 0)),
        ],
        out_specs=pl.BlockSpec((1, C, N), lambda b: (b, 0, 0)),
        out_shape=jax.ShapeDtypeStruct((B, C, N), jnp.float32),
        compiler_params=pltpu.CompilerParams(
            dimension_semantics=("parallel",),
            vmem_limit_bytes=50 * 1024 * 1024,
        ),
    )(fm, coords, scale)
    return jnp.transpose(out, (0, 2, 1))
